# baseline (speedup 1.0000x reference)
import functools

import jax
import jax.numpy as jnp
from jax import lax
from jax.experimental import pallas as pl
from jax.experimental.pallas import tpu as pltpu
from jax.experimental.pallas import tpu_sc as plsc

_V = 50257
_D = 512
_PAD_ID = 50256
_SCALE = float(_D) ** 0.5

_NUM_WORKERS = 32
_SUB = 64
_NSUB = 4
_CHUNK = _SUB * _NSUB
_LANES = 16


def _emb_body(ids_hbm, tok_hbm, pos_hbm, x_hbm, mask_hbm,
              ids_v, rows_v, pos_v, mask_v, sem):
    nc = plsc.get_sparse_core_info().num_cores
    wid = lax.axis_index("s") * nc + lax.axis_index("c")
    base = wid * _CHUNK
    t0 = lax.rem(base, 2048)

    pltpu.sync_copy(ids_hbm.at[wid], ids_v)

    def mask_row(j, _):
        def mask_vec(k, _):
            v = ids_v[j, pl.ds(k * _LANES, _LANES)]
            mask_v[j, pl.ds(k * _LANES, _LANES)] = jnp.where(
                v != _PAD_ID, jnp.int32(1), jnp.int32(0))
            return 0
        return lax.fori_loop(0, _SUB // _LANES, mask_vec, 0)
    lax.fori_loop(0, _NSUB, mask_row, 0)
    pltpu.sync_copy(mask_v, mask_hbm.at[wid])

    for j in range(_NSUB):
        pltpu.async_copy(tok_hbm.at[ids_v.at[j]], rows_v, sem).wait()
        pltpu.sync_copy(pos_hbm.at[pl.ds(t0 + j * _SUB, _SUB)], pos_v)

        def row(r, _):
            for c in range(_D // _LANES):
                sl = pl.ds(c * _LANES, _LANES)
                rows_v[r, sl] = rows_v[r, sl] * _SCALE + pos_v[r, sl]
            return 0
        lax.fori_loop(0, _SUB, row, 0)

        pltpu.sync_copy(rows_v, x_hbm.at[pl.ds(base + j * _SUB, _SUB)])


@jax.jit
def _embed(ids_grouped, tok_emb, pos_emb):
    mesh = plsc.VectorSubcoreMesh(core_axis_name="c", subcore_axis_name="s")
    f = pl.kernel(
        _emb_body,
        out_type=(
            jax.ShapeDtypeStruct((_NUM_WORKERS * _CHUNK, _D), jnp.float32),
            jax.ShapeDtypeStruct((_NUM_WORKERS, _NSUB, _SUB), jnp.int32),
        ),
        mesh=mesh,
        scratch_types=[
            pltpu.VMEM((_NSUB, _SUB), jnp.int32),
            pltpu.VMEM((_SUB, _D), jnp.float32),
            pltpu.VMEM((_SUB, _D), jnp.float32),
            pltpu.VMEM((_NSUB, _SUB), jnp.int32),
            pltpu.SemaphoreType.DMA,
        ],
    )
    return f(ids_grouped, tok_emb, pos_emb)


def kernel(token_ids, tok_emb, pos_emb):
    B, T = token_ids.shape
    ids_grouped = token_ids.reshape(_NUM_WORKERS, _NSUB, _SUB)
    x_flat, mask_i32 = _embed(ids_grouped, tok_emb, pos_emb)
    x = x_flat.reshape(B, T, _D)
    attn_mask = mask_i32.reshape(B, T).astype(bool)[:, None, None, :]
    return (x, attn_mask)


# speedup vs baseline: 1.2067x; 1.2067x over previous
import jax
import jax.numpy as jnp
from jax import lax
from jax.experimental import pallas as pl
from jax.experimental.pallas import tpu as pltpu
from jax.experimental.pallas import tpu_sc as plsc

_V = 50257
_D = 512
_T = 2048
_B = 4
_PAD_ID = 50256
_SCALE = float(_D) ** 0.5

_NUM_WORKERS = 32
_TW = _T // _NUM_WORKERS
_SUB = 32
_NCHUNK = _B * _TW // _SUB
_LANES = 16


def _emb_body(ids_hbm, tok_hbm, pos_hbm, x_hbm, mask_hbm,
              ids_v, pos_v, in_v, out_v, mask_v, gsems, wsems):
    nc = plsc.get_sparse_core_info().num_cores
    wid = lax.axis_index("s") * nc + lax.axis_index("c")
    t0 = wid * _TW

    pos_cp = pltpu.async_copy(pos_hbm.at[pl.ds(t0, _TW)], pos_v, gsems.at[2])
    pltpu.sync_copy(ids_hbm.at[wid], ids_v)

    def gather_idx(c):
        b, h = divmod(c, _TW // _SUB)
        return ids_v.at[b, pl.ds(h * _SUB, _SUB)]

    def out_slice(c):
        b, h = divmod(c, _TW // _SUB)
        return pl.ds(b * _T + t0 + h * _SUB, _SUB)

    gd = {}
    for c in range(2):
        gd[c] = pltpu.async_copy(tok_hbm.at[gather_idx(c)], in_v.at[c % 2],
                                 gsems.at[c % 2])

    def mask_vec(k, _):
        b, h = divmod(k, _TW // _LANES)
        sl = pl.ds(h * _LANES, _LANES)
        v = ids_v[b, sl]
        mask_v[b, sl] = jnp.where(v != _PAD_ID, jnp.int32(1), jnp.int32(0))
        return 0
    lax.fori_loop(0, _B * _TW // _LANES, mask_vec, 0)
    pltpu.sync_copy(mask_v, mask_hbm.at[wid])
    pos_cp.wait()

    wd = {}
    for c in range(_NCHUNK):
        gd.pop(c).wait()
        if c >= 2:
            wd.pop(c - 2).wait()

        ph = (c % (_TW // _SUB)) * _SUB

        def row(r, _):
            for k in range(_D // _LANES):
                sl = pl.ds(k * _LANES, _LANES)
                out_v[c % 2, r, sl] = (in_v[c % 2, r, sl] * _SCALE
                                       + pos_v[ph + r, sl])
            return 0
        lax.fori_loop(0, _SUB, row, 0)

        if c + 2 < _NCHUNK:
            gd[c + 2] = pltpu.async_copy(tok_hbm.at[gather_idx(c + 2)],
                                         in_v.at[c % 2], gsems.at[c % 2])
        wd[c] = pltpu.async_copy(out_v.at[c % 2], x_hbm.at[out_slice(c)],
                                 wsems.at[c % 2])
    wd.pop(_NCHUNK - 2).wait()
    wd.pop(_NCHUNK - 1).wait()


@jax.jit
def _embed(ids_grouped, tok_emb, pos_emb):
    mesh = plsc.VectorSubcoreMesh(core_axis_name="c", subcore_axis_name="s")
    f = pl.kernel(
        _emb_body,
        out_type=(
            jax.ShapeDtypeStruct((_B * _T, _D), jnp.float32),
            jax.ShapeDtypeStruct((_NUM_WORKERS, _B, _TW), jnp.int32),
        ),
        mesh=mesh,
        scratch_types=[
            pltpu.VMEM((_B, _TW), jnp.int32),
            pltpu.VMEM((_TW, _D), jnp.float32),
            pltpu.VMEM((2, _SUB, _D), jnp.float32),
            pltpu.VMEM((2, _SUB, _D), jnp.float32),
            pltpu.VMEM((_B, _TW), jnp.int32),
            pltpu.SemaphoreType.DMA((3,)),
            pltpu.SemaphoreType.DMA((2,)),
        ],
    )
    return f(ids_grouped, tok_emb, pos_emb)


def kernel(token_ids, tok_emb, pos_emb):
    B, T = token_ids.shape
    ids_grouped = token_ids.reshape(B, _NUM_WORKERS, _TW).transpose(1, 0, 2)
    x_flat, mask_g = _embed(ids_grouped, tok_emb, pos_emb)
    x = x_flat.reshape(B, T, _D)
    mask = mask_g.transpose(1, 0, 2).reshape(B, T)
    attn_mask = mask.astype(bool)[:, None, None, :]
    return (x, attn_mask)


# speedup vs baseline: 1.2679x; 1.0507x over previous
import jax
import jax.numpy as jnp
from jax import lax
from jax.experimental import pallas as pl
from jax.experimental.pallas import tpu as pltpu
from jax.experimental.pallas import tpu_sc as plsc

_V = 50257
_D = 512
_T = 2048
_B = 4
_PAD_ID = 50256
_SCALE = float(_D) ** 0.5

_NUM_WORKERS = 32
_TW = _T // _NUM_WORKERS
_ST = 8
_NCHUNK = _TW // _ST
_ROWS = _B * _ST
_NBUF = 4
_LANES = 16


def _emb_body(ids_hbm, tok_hbm, pos_hbm, x_hbm, mask_hbm,
              ids_g, mask_v, pos_v, buf,
              idsem, possem, gsems, wsems):
    nc = plsc.get_sparse_core_info().num_cores
    wid = lax.axis_index("s") * nc + lax.axis_index("c")
    t0 = wid * _TW

    id_cp = pltpu.async_copy(ids_hbm.at[wid], ids_g, idsem)
    pos_cp = pltpu.async_copy(pos_hbm.at[pl.ds(t0, _TW)], pos_v, possem)
    id_cp.wait()

    gd = {}
    for c in range(2):
        gd[c] = pltpu.async_copy(tok_hbm.at[ids_g.at[c]], buf.at[c % _NBUF],
                                 gsems.at[c % _NBUF])

    for h in range(_NCHUNK):
        for half in range(_ROWS // _LANES):
            sl = pl.ds(half * _LANES, _LANES)
            v = ids_g[h, sl]
            mask_v[h, sl] = jnp.where(v != _PAD_ID, jnp.int32(1), jnp.int32(0))
    pltpu.sync_copy(mask_v, mask_hbm.at[wid])
    pos_cp.wait()

    wd = {}
    for c in range(_NCHUNK):
        gd.pop(c).wait()
        buf_c = buf.at[c % _NBUF]

        def row(t, _):
            for k in range(_D // _LANES):
                sl = pl.ds(k * _LANES, _LANES)
                pv = pos_v[c * _ST + t, sl]
                for b in range(_B):
                    r = b * _ST + t
                    buf_c[r, sl] = buf_c[r, sl] * _SCALE + pv
            return 0
        lax.fori_loop(0, _ST, row, 0)

        wd[c] = [pltpu.async_copy(
            buf_c.at[pl.ds(b * _ST, _ST)],
            x_hbm.at[pl.ds(b * _T + t0 + c * _ST, _ST)],
            wsems.at[c % _NBUF]) for b in range(_B)]
        if c + 2 < _NCHUNK:
            if c >= 2:
                for cp in wd.pop(c - 2):
                    cp.wait()
            gd[c + 2] = pltpu.async_copy(tok_hbm.at[ids_g.at[c + 2]],
                                         buf.at[(c + 2) % _NBUF],
                                         gsems.at[(c + 2) % _NBUF])
    for c in sorted(wd):
        for cp in wd[c]:
            cp.wait()


@jax.jit
def _embed(ids_g, tok_emb, pos_emb):
    mesh = plsc.VectorSubcoreMesh(core_axis_name="c", subcore_axis_name="s")
    f = pl.kernel(
        _emb_body,
        out_type=(
            jax.ShapeDtypeStruct((_B * _T, _D), jnp.float32),
            jax.ShapeDtypeStruct((_NUM_WORKERS, _NCHUNK, _ROWS), jnp.int32),
        ),
        mesh=mesh,
        scratch_types=[
            pltpu.VMEM((_NCHUNK, _ROWS), jnp.int32),
            pltpu.VMEM((_NCHUNK, _ROWS), jnp.int32),
            pltpu.VMEM((_TW, _D), jnp.float32),
            pltpu.VMEM((_NBUF, _ROWS, _D), jnp.float32),
            pltpu.SemaphoreType.DMA,
            pltpu.SemaphoreType.DMA,
            pltpu.SemaphoreType.DMA((_NBUF,)),
            pltpu.SemaphoreType.DMA((_NBUF,)),
        ],
    )
    return f(ids_g, tok_emb, pos_emb)


def kernel(token_ids, tok_emb, pos_emb):
    B, T = token_ids.shape
    ids_g = (token_ids.reshape(B, _NUM_WORKERS, _NCHUNK, _ST)
             .transpose(1, 2, 0, 3).reshape(_NUM_WORKERS, _NCHUNK, _ROWS))
    x_flat, mask_g = _embed(ids_g, tok_emb, pos_emb)
    x = x_flat.reshape(B, T, _D)
    mask = (mask_g.reshape(_NUM_WORKERS, _NCHUNK, _B, _ST)
            .transpose(2, 0, 1, 3).reshape(B, T))
    attn_mask = mask.astype(bool)[:, None, None, :]
    return (x, attn_mask)
